# baseline (speedup 1.0000x reference)
import functools

import jax
import jax.numpy as jnp
from jax import lax
from jax.experimental import pallas as pl
from jax.experimental.pallas import tpu as pltpu
from jax.experimental.pallas import tpu_sc as plsc

V = 1000000
B = 16384
S = 50
D = 64
NL = 5
NLP = 16

NC = 2
NS = 16
L = 16
NW = NC * NS
BPW = B // NW
CH = 32
ROWS = CH * S
NCHUNK = BPW // CH
NBUF = 2


_RB = 8192
_NSEG = 8
_CB = _NSEG * _RB
_GRID = -(-V // _CB)
_PROWS = _GRID * _RB
_TABV = _PROWS * _NSEG


def _fold_body(x_ref, ws_ref, o_ref):
    xt = x_ref[...].astype(jnp.bfloat16).T
    for s in range(_NSEG):
        y = jnp.dot(xt[s * _RB:(s + 1) * _RB, :],
                    ws_ref[s * D:(s + 1) * D, :],
                    preferred_element_type=jnp.float32)
        o_ref[:, s * NLP:(s + 1) * NLP] = y[:, s * NLP:(s + 1) * NLP]


_fold = pl.pallas_call(
    _fold_body,
    grid=(_GRID,),
    in_specs=[
        pl.BlockSpec((D, _CB), lambda i: (0, i)),
        pl.BlockSpec((_NSEG * D, _NSEG * NLP), lambda i: (0, 0)),
    ],
    out_specs=pl.BlockSpec((_RB, _NSEG * NLP), lambda i: (i, 0)),
    out_shape=jax.ShapeDtypeStruct((_PROWS, _NSEG * NLP), jnp.float32),
    compiler_params=pltpu.CompilerParams(fuse_transposed_lhs_in_matmul=True),
)


def _pool_body(idx_hbm, tab_hbm, out_hbm, idx_v, out_v, buf0, buf1, sem0,
               sem1):
    wid = lax.axis_index("s") * NC + lax.axis_index("c")
    pltpu.sync_copy(idx_hbm.at[wid], idx_v)

    bufs = (buf0, buf1)
    sems = (sem0, sem1)

    def _idx_slice(g):
        return idx_v.at[pl.ds(g * ROWS, ROWS)]

    for k in range(NBUF):
        pltpu.async_copy(tab_hbm.at[_idx_slice(k)], bufs[k], sems[k])

    @pl.loop(0, NCHUNK, step=NBUF)
    def _chunks(g0):
        for k in range(NBUF):
            g = g0 + k
            buf = bufs[k]
            pltpu.make_async_copy(tab_hbm.at[_idx_slice(g)], buf,
                                  sems[k]).wait()

            def _elem(e, _):
                base = e * S
                zero = jnp.zeros((L,), jnp.float32)
                acc = [zero, zero, zero, zero]
                for r in range(S):
                    acc[r % 4] = acc[r % 4] + buf[base + r, pl.ds(0, L)]
                orow = g * CH + e
                out_v[orow, pl.ds(0, L)] = (acc[0] + acc[1]) + (acc[2] +
                                                                acc[3])
                return 0

            lax.fori_loop(0, CH, _elem, 0)

            @pl.when(g + NBUF < NCHUNK)
            def _():
                pltpu.async_copy(tab_hbm.at[_idx_slice(g + NBUF)], buf,
                                 sems[k])

    pltpu.sync_copy(out_v, out_hbm.at[pl.ds(wid * BPW, BPW)])


@functools.cache
def _make_pool():
    return pl.kernel(
        _pool_body,
        out_type=jax.ShapeDtypeStruct((B, NLP), jnp.float32),
        mesh=plsc.VectorSubcoreMesh(core_axis_name="c", subcore_axis_name="s",
                                    num_cores=NC, num_subcores=NS),
        compiler_params=pltpu.CompilerParams(use_tc_tiling_on_sc=False),
        scratch_types=[
            pltpu.VMEM((NCHUNK * ROWS,), jnp.int32),
            pltpu.VMEM((BPW, NLP), jnp.float32),
            pltpu.VMEM((ROWS, NLP), jnp.float32),
            pltpu.VMEM((ROWS, NLP), jnp.float32),
            pltpu.SemaphoreType.DMA,
            pltpu.SemaphoreType.DMA,
        ],
    )


_BT = 2048


def _cls_body(x_ref, b_ref, o_ref):
    logits = x_ref[:, 0:NL] * (1.0 / S) + b_ref[...]
    m = jnp.max(logits, axis=1, keepdims=True)
    ex = jnp.exp(logits - m)
    lse = jnp.log(jnp.sum(ex, axis=1, keepdims=True)) + m
    o_ref[...] = logits - lse


_cls = pl.pallas_call(
    _cls_body,
    grid=(B // _BT,),
    in_specs=[
        pl.BlockSpec((_BT, NLP), lambda i: (i, 0)),
        pl.BlockSpec((1, NL), lambda i: (0, 0)),
    ],
    out_specs=pl.BlockSpec((_BT, NL), lambda i: (i, 0)),
    out_shape=jax.ShapeDtypeStruct((B, NL), jnp.float32),
)


def kernel(batch_input, emb, W, b):
    wp = jnp.pad(W, ((0, 0), (0, NLP - NL)))
    ws = jnp.zeros((_NSEG * D, _NSEG * NLP), jnp.bfloat16)
    for s in range(_NSEG):
        ws = ws.at[s * D:(s + 1) * D,
                   s * NLP:(s + 1) * NLP].set(wp.astype(jnp.bfloat16))
    packed = _fold(emb.T, ws)
    tab = packed.reshape(_TABV, NLP)
    i = batch_input
    idx = (((i // _CB) * _RB + i % _RB) * _NSEG
           + (i // _RB) % _NSEG).reshape(NW, NCHUNK * ROWS)
    pooled = _make_pool()(idx, tab)
    return _cls(pooled, b.reshape(1, NL))


# speedup vs baseline: 1.1172x; 1.1172x over previous
import functools

import jax
import jax.numpy as jnp
from jax import lax
from jax.experimental import pallas as pl
from jax.experimental.pallas import tpu as pltpu
from jax.experimental.pallas import tpu_sc as plsc

V = 1000000
B = 16384
S = 50
D = 64
NL = 5
NLP = 16

NC = 2
NS = 16
L = 16
NW = NC * NS
BPW = B // NW
CH = 32
ROWS = CH * S
NCHUNK = BPW // CH
NBUF = 2


_RB = 4096
_SH2 = 12
_NSEG = 8
_CB = _NSEG * _RB
_SH1 = 15
_GRID = -(-V // _CB)
_PROWS = _GRID * _RB
_TABV = _PROWS * _NSEG


def _fold_body(x_ref, ws_ref, o_ref):
    xt = x_ref[...].astype(jnp.bfloat16).T
    for s in range(_NSEG):
        y = jnp.dot(xt[s * _RB:(s + 1) * _RB, :],
                    ws_ref[s * D:(s + 1) * D, :],
                    preferred_element_type=jnp.float32)
        o_ref[:, s * NLP:(s + 1) * NLP] = y[:, s * NLP:(s + 1) * NLP]


_fold = pl.pallas_call(
    _fold_body,
    grid=(_GRID,),
    in_specs=[
        pl.BlockSpec((D, _CB), lambda i: (0, i)),
        pl.BlockSpec((_NSEG * D, _NSEG * NLP), lambda i: (0, 0)),
    ],
    out_specs=pl.BlockSpec((_RB, _NSEG * NLP), lambda i: (i, 0)),
    out_shape=jax.ShapeDtypeStruct((_PROWS, _NSEG * NLP), jnp.float32),
    compiler_params=pltpu.CompilerParams(fuse_transposed_lhs_in_matmul=True),
)


def _pool_body(tok_hbm, tab_hbm, out_hbm, tok_v, out_v, idx0, idx1, buf0,
               buf1, sem0, sem1):
    wid = lax.axis_index("s") * NC + lax.axis_index("c")
    pltpu.sync_copy(tok_hbm.at[:, pl.ds(wid * BPW, BPW)], tok_v)

    idxs = (idx0, idx1)
    bufs = (buf0, buf1)
    sems = (sem0, sem1)

    def _xform(c, idx_buf):
        def _t(t, _):
            for jj in range(CH // L):
                i = tok_v[t, pl.ds(c * CH + jj * L, L)]
                v = (((i >> _SH1) << _SH2) | (i & (_RB - 1)))
                v = (v << 3) | ((i >> _SH2) & (_NSEG - 1))
                idx_buf[pl.ds(t * CH + jj * L, L)] = v
            return 0

        lax.fori_loop(0, S, _t, 0)

    for k in range(NBUF):
        _xform(k, idxs[k])
        pltpu.async_copy(tab_hbm.at[idxs[k]], bufs[k], sems[k])

    @pl.loop(0, NCHUNK, step=NBUF)
    def _chunks(g0):
        for k in range(NBUF):
            g = g0 + k
            buf = bufs[k]
            pltpu.make_async_copy(tab_hbm.at[idxs[k]], buf, sems[k]).wait()

            def _elem(e, _):
                zero = jnp.zeros((L,), jnp.float32)
                acc = [zero, zero, zero, zero]
                for t in range(S):
                    acc[t % 4] = acc[t % 4] + buf[t * CH + e, pl.ds(0, L)]
                orow = g * CH + e
                out_v[orow, pl.ds(0, L)] = (acc[0] + acc[1]) + (acc[2] +
                                                                acc[3])
                return 0

            lax.fori_loop(0, CH, _elem, 0)

            @pl.when(g + NBUF < NCHUNK)
            def _():
                _xform(g + NBUF, idxs[k])
                pltpu.async_copy(tab_hbm.at[idxs[k]], buf, sems[k])

    pltpu.sync_copy(out_v, out_hbm.at[pl.ds(wid * BPW, BPW)])


@functools.cache
def _make_pool():
    return pl.kernel(
        _pool_body,
        out_type=jax.ShapeDtypeStruct((B, NLP), jnp.float32),
        mesh=plsc.VectorSubcoreMesh(core_axis_name="c", subcore_axis_name="s",
                                    num_cores=NC, num_subcores=NS),
        compiler_params=pltpu.CompilerParams(use_tc_tiling_on_sc=False),
        scratch_types=[
            pltpu.VMEM((S, BPW), jnp.int32),
            pltpu.VMEM((BPW, NLP), jnp.float32),
            pltpu.VMEM((ROWS,), jnp.int32),
            pltpu.VMEM((ROWS,), jnp.int32),
            pltpu.VMEM((ROWS, NLP), jnp.float32),
            pltpu.VMEM((ROWS, NLP), jnp.float32),
            pltpu.SemaphoreType.DMA,
            pltpu.SemaphoreType.DMA,
        ],
    )


_BT = 2048


def _cls_body(x_ref, b_ref, o_ref):
    logits = x_ref[:, 0:NL] * (1.0 / S) + b_ref[...]
    m = jnp.max(logits, axis=1, keepdims=True)
    ex = jnp.exp(logits - m)
    lse = jnp.log(jnp.sum(ex, axis=1, keepdims=True)) + m
    o_ref[...] = (logits - lse).T


_cls = pl.pallas_call(
    _cls_body,
    grid=(B // _BT,),
    in_specs=[
        pl.BlockSpec((_BT, NLP), lambda i: (i, 0)),
        pl.BlockSpec((1, NL), lambda i: (0, 0)),
    ],
    out_specs=pl.BlockSpec((NL, _BT), lambda i: (0, i)),
    out_shape=jax.ShapeDtypeStruct((NL, B), jnp.float32),
)


def kernel(batch_input, emb, W, b):
    wp = jnp.pad(W, ((0, 0), (0, NLP - NL)))
    ws = jnp.zeros((_NSEG * D, _NSEG * NLP), jnp.bfloat16)
    for s in range(_NSEG):
        ws = ws.at[s * D:(s + 1) * D,
                   s * NLP:(s + 1) * NLP].set(wp.astype(jnp.bfloat16))
    packed = _fold(emb.T, ws)
    tab = packed.reshape(_TABV, NLP)
    pooled = _make_pool()(batch_input.T, tab)
    return _cls(pooled, b.reshape(1, NL)).T


# speedup vs baseline: 1.1658x; 1.0435x over previous
import functools

import jax
import jax.numpy as jnp
from jax import lax
from jax.experimental import pallas as pl
from jax.experimental.pallas import tpu as pltpu
from jax.experimental.pallas import tpu_sc as plsc

V = 1000000
B = 16384
S = 50
D = 64
NL = 5
NLP = 16

NC = 2
NS = 16
L = 16
NW = NC * NS
BPW = B // NW
CH = 32
ROWS = CH * S
NCHUNK = BPW // CH
NBUF = 3


_RB = 4096
_SH2 = 12
_NSEG = 8
_CB = _NSEG * _RB
_SH1 = 15
_GRID = -(-V // _CB)
_PROWS = _GRID * _RB
_TABV = _PROWS * _NSEG


def _fold_body(x_ref, ws_ref, o_ref):
    xt = x_ref[...].astype(jnp.bfloat16).T
    for s in range(_NSEG):
        y = jnp.dot(xt[s * _RB:(s + 1) * _RB, :],
                    ws_ref[s * D:(s + 1) * D, :],
                    preferred_element_type=jnp.float32)
        o_ref[:, s * NLP:(s + 1) * NLP] = y[:, s * NLP:(s + 1) * NLP]


_fold = pl.pallas_call(
    _fold_body,
    grid=(_GRID,),
    in_specs=[
        pl.BlockSpec((D, _CB), lambda i: (0, i)),
        pl.BlockSpec((_NSEG * D, _NSEG * NLP), lambda i: (0, 0)),
    ],
    out_specs=pl.BlockSpec((_RB, _NSEG * NLP), lambda i: (i, 0)),
    out_shape=jax.ShapeDtypeStruct((_PROWS, _NSEG * NLP), jnp.float32),
    compiler_params=pltpu.CompilerParams(fuse_transposed_lhs_in_matmul=True),
)


def _pool_body(tok_hbm, tab_hbm, out_hbm, tok_v, out_v, idx0, idx1, idx2,
               buf0, buf1, buf2, sem0, sem1, sem2):
    wid = lax.axis_index("s") * NC + lax.axis_index("c")
    pltpu.sync_copy(tok_hbm.at[:, pl.ds(wid * BPW, BPW)], tok_v)

    idxs = (idx0, idx1, idx2)
    bufs = (buf0, buf1, buf2)
    sems = (sem0, sem1, sem2)

    def _xform(c, idx_buf):
        def _t(t, _):
            for jj in range(CH // L):
                i = tok_v[t, pl.ds(c * CH + jj * L, L)]
                v = (((i >> _SH1) << _SH2) | (i & (_RB - 1)))
                v = (v << 3) | ((i >> _SH2) & (_NSEG - 1))
                idx_buf[pl.ds(t * CH + jj * L, L)] = v
            return 0

        lax.fori_loop(0, S, _t, 0)

    for k in range(NBUF):
        _xform(k, idxs[k])
        pltpu.async_copy(tab_hbm.at[idxs[k]], bufs[k], sems[k])

    @pl.loop(0, NCHUNK, step=NBUF)
    def _chunks(g0):
        for k in range(NBUF):
            g = g0 + k
            buf = bufs[k]

            @pl.when(g < NCHUNK)
            def _():
                pltpu.make_async_copy(tab_hbm.at[idxs[k]], buf,
                                      sems[k]).wait()

                def _elem(e, _):
                    zero = jnp.zeros((L,), jnp.float32)
                    acc = [zero, zero, zero, zero]
                    for t in range(S):
                        acc[t % 4] = acc[t % 4] + buf[t * CH + e,
                                                      pl.ds(0, L)]
                    orow = g * CH + e
                    out_v[orow, pl.ds(0, L)] = (acc[0] + acc[1]) + (acc[2] +
                                                                    acc[3])
                    return 0

                lax.fori_loop(0, CH, _elem, 0)

                @pl.when(g + NBUF < NCHUNK)
                def _():
                    _xform(g + NBUF, idxs[k])
                    pltpu.async_copy(tab_hbm.at[idxs[k]], buf, sems[k])

    pltpu.sync_copy(out_v, out_hbm.at[pl.ds(wid * BPW, BPW)])


@functools.cache
def _make_pool():
    return pl.kernel(
        _pool_body,
        out_type=jax.ShapeDtypeStruct((B, NLP), jnp.float32),
        mesh=plsc.VectorSubcoreMesh(core_axis_name="c", subcore_axis_name="s",
                                    num_cores=NC, num_subcores=NS),
        compiler_params=pltpu.CompilerParams(use_tc_tiling_on_sc=False),
        scratch_types=[
            pltpu.VMEM((S, BPW), jnp.int32),
            pltpu.VMEM((BPW, NLP), jnp.float32),
            pltpu.VMEM((ROWS,), jnp.int32),
            pltpu.VMEM((ROWS,), jnp.int32),
            pltpu.VMEM((ROWS,), jnp.int32),
            pltpu.VMEM((ROWS, NLP), jnp.float32),
            pltpu.VMEM((ROWS, NLP), jnp.float32),
            pltpu.VMEM((ROWS, NLP), jnp.float32),
            pltpu.SemaphoreType.DMA,
            pltpu.SemaphoreType.DMA,
            pltpu.SemaphoreType.DMA,
        ],
    )


_BT = 2048


def _cls_body(x_ref, b_ref, o_ref):
    logits = x_ref[:, 0:NL] * (1.0 / S) + b_ref[...]
    m = jnp.max(logits, axis=1, keepdims=True)
    ex = jnp.exp(logits - m)
    lse = jnp.log(jnp.sum(ex, axis=1, keepdims=True)) + m
    o_ref[...] = (logits - lse).T


_cls = pl.pallas_call(
    _cls_body,
    grid=(B // _BT,),
    in_specs=[
        pl.BlockSpec((_BT, NLP), lambda i: (i, 0)),
        pl.BlockSpec((1, NL), lambda i: (0, 0)),
    ],
    out_specs=pl.BlockSpec((NL, _BT), lambda i: (0, i)),
    out_shape=jax.ShapeDtypeStruct((NL, B), jnp.float32),
)


def kernel(batch_input, emb, W, b):
    wp = jnp.pad(W, ((0, 0), (0, NLP - NL)))
    ws = jnp.zeros((_NSEG * D, _NSEG * NLP), jnp.bfloat16)
    for s in range(_NSEG):
        ws = ws.at[s * D:(s + 1) * D,
                   s * NLP:(s + 1) * NLP].set(wp.astype(jnp.bfloat16))
    packed = _fold(emb.T, ws)
    tab = packed.reshape(_TABV, NLP)
    pooled = _make_pool()(batch_input.T, tab)
    return _cls(pooled, b.reshape(1, NL)).T
